# baseline (speedup 1.0000x reference)
import jax
import jax.numpy as jnp
from jax.experimental import pallas as pl
from jax.experimental.pallas import tpu as pltpu

TOK = 16384
DM = 2048
NE = 64
CT = 512
DEPTH = 8
NCH = TOK // CT
NWIN = NCH // DEPTH
WTOK = DEPTH * CT


def _gate_kernel(x_hbm, wt_ref, b_ref, o_hbm, *scr):
    xbufs = scr[:DEPTH]
    obufs = scr[DEPTH:2 * DEPTH]
    in_sems = scr[2 * DEPTH:3 * DEPTH]
    out_sems = scr[3 * DEPTH:4 * DEPTH]
    wb = wt_ref[...].astype(jnp.bfloat16)
    bias = b_ref[...]

    def in_copy(off, k):
        return pltpu.make_async_copy(
            x_hbm.at[pl.ds(off, CT), :], xbufs[k], in_sems[k])

    def out_copy(off, k):
        return pltpu.make_async_copy(
            obufs[k], o_hbm.at[pl.ds(off, CT), :], out_sems[k])

    for k in range(DEPTH):
        in_copy(k * CT, k).start()

    def body(it, carry):
        base = it * WTOK
        for k in range(DEPTH):
            in_copy(base + k * CT, k).wait()

            @pl.when(it > 0)
            def _():
                out_copy((it - 1) * WTOK + k * CT, k).wait()

            xb = xbufs[k][...].astype(jnp.bfloat16)
            logits = jnp.dot(xb, wb, preferred_element_type=jnp.float32) + bias
            m = jnp.max(logits, axis=-1, keepdims=True)
            e = jnp.exp(logits - m)
            obufs[k][...] = e / jnp.sum(e, axis=-1, keepdims=True)
            out_copy(base + k * CT, k).start()

            @pl.when(it < NWIN - 1)
            def _():
                in_copy(base + WTOK + k * CT, k).start()
        return carry

    jax.lax.fori_loop(0, NWIN, body, 0)
    for k in range(DEPTH):
        out_copy((NWIN - 1) * WTOK + k * CT, k).wait()


def kernel(x, W, b):
    scratch = (
        [pltpu.VMEM((CT, DM), jnp.float32) for _ in range(DEPTH)]
        + [pltpu.VMEM((CT, NE), jnp.float32) for _ in range(DEPTH)]
        + [pltpu.SemaphoreType.DMA for _ in range(2 * DEPTH)]
    )
    return pl.pallas_call(
        _gate_kernel,
        in_specs=[
            pl.BlockSpec(memory_space=pltpu.MemorySpace.HBM),
            pl.BlockSpec(memory_space=pltpu.MemorySpace.VMEM),
            pl.BlockSpec(memory_space=pltpu.MemorySpace.VMEM),
        ],
        out_specs=pl.BlockSpec(memory_space=pltpu.MemorySpace.HBM),
        out_shape=jax.ShapeDtypeStruct((TOK, NE), jnp.float32),
        scratch_shapes=scratch,
    )(x, W.T, b.reshape(1, NE))


# speedup vs baseline: 1.1180x; 1.1180x over previous
import jax
import jax.numpy as jnp
from jax.experimental import pallas as pl
from jax.experimental.pallas import tpu as pltpu

BT = 1024


def _gate_kernel(x_ref, w_ref, b_ref, o_ref, wbt):
    @pl.when(pl.program_id(0) == 0)
    def _():
        wbt[...] = w_ref[...].T.astype(jnp.bfloat16)

    xb = x_ref[...].astype(jnp.bfloat16)
    logits = jnp.dot(xb, wbt[...], preferred_element_type=jnp.float32) + b_ref[...]
    m = jnp.max(logits, axis=-1, keepdims=True)
    e = jnp.exp(logits - m)
    o_ref[...] = e / jnp.sum(e, axis=-1, keepdims=True)


def kernel(x, W, b):
    T, D = x.shape
    E = W.shape[0]
    return pl.pallas_call(
        _gate_kernel,
        grid=(T // BT,),
        in_specs=[
            pl.BlockSpec((BT, D), lambda i: (i, 0)),
            pl.BlockSpec((E, D), lambda i: (0, 0)),
            pl.BlockSpec((E,), lambda i: (0,)),
        ],
        out_specs=pl.BlockSpec((BT, E), lambda i: (i, 0)),
        out_shape=jax.ShapeDtypeStruct((T, E), jnp.float32),
        scratch_shapes=[pltpu.VMEM((D, E), jnp.bfloat16)],
    )(x, W, b)
